# baseline (speedup 1.0000x reference)
import jax
import jax.numpy as jnp
from jax.experimental import pallas as pl

B, N, F = 16, 4096, 512
E = 64
HI = 2048
RB = 4096
NCOPY = N // RB
NBLK = NCOPY + 1


def _body(x_ref, lo_ref, hi_ref, out_ref):
    j = pl.program_id(1)

    @pl.when(j < NCOPY)
    def _copy():
        out_ref[...] = x_ref[...]

    @pl.when(j == NCOPY)
    def _tail():
        out_ref[0, :E, :] = 0.5 * (lo_ref[0] + hi_ref[0])


def kernel(inputs):
    grid = (B, NBLK)
    return pl.pallas_call(
        _body,
        grid=grid,
        in_specs=[
            pl.BlockSpec((1, RB, F), lambda b, j: (b, jnp.minimum(j, NCOPY - 1), 0)),
            pl.BlockSpec((1, E, F), lambda b, j: (b, 0, 0)),
            pl.BlockSpec((1, E, F), lambda b, j: (b, HI // E, 0)),
        ],
        out_specs=pl.BlockSpec((1, RB, F), lambda b, j: (b, j, 0)),
        out_shape=jax.ShapeDtypeStruct((B, N + E, F), inputs.dtype),
    )(inputs, inputs, inputs)


# speedup vs baseline: 1.3243x; 1.3243x over previous
import jax
import jax.numpy as jnp
from jax.experimental import pallas as pl

B, N, F = 16, 4096, 512
E = 64
HI = 2048
RB = 2080
NBLK = (N + E) // RB
TAIL_COPY = N - (NBLK - 1) * RB


def _body(x_ref, lo_ref, hi_ref, out_ref):
    j = pl.program_id(1)

    @pl.when(j < NBLK - 1)
    def _copy():
        out_ref[...] = x_ref[...]

    @pl.when(j == NBLK - 1)
    def _tail():
        out_ref[0, :TAIL_COPY, :] = x_ref[0, :TAIL_COPY, :]
        out_ref[0, TAIL_COPY:, :] = 0.5 * (lo_ref[0] + hi_ref[0])


def kernel(inputs):
    grid = (B, NBLK)
    return pl.pallas_call(
        _body,
        grid=grid,
        in_specs=[
            pl.BlockSpec((1, RB, F), lambda b, j: (b, j, 0)),
            pl.BlockSpec((1, E, F), lambda b, j: (b, 0, 0)),
            pl.BlockSpec((1, E, F), lambda b, j: (b, HI // E, 0)),
        ],
        out_specs=pl.BlockSpec((1, RB, F), lambda b, j: (b, j, 0)),
        out_shape=jax.ShapeDtypeStruct((B, N + E, F), inputs.dtype),
    )(inputs, inputs, inputs)
